# baseline (speedup 1.0000x reference)
import jax
import jax.numpy as jnp
from jax.experimental import pallas as pl
from jax.experimental.pallas import tpu as pltpu

_KBLOCK = 256
_LANES = 128


def _loss_kernel(x_hbm, mu_hbm, out_ref, x_v, mu_v, dots_v, sem_x, sem_mu):
    B, G = x_v.shape
    K = mu_v.shape[1]
    gh = G // 2

    cp_x = pltpu.make_async_copy(x_hbm, x_v, sem_x)
    cp_mu0 = pltpu.make_async_copy(
        mu_hbm.at[pl.ds(0, gh), :], mu_v.at[pl.ds(0, gh), :], sem_mu.at[0])
    cp_mu1 = pltpu.make_async_copy(
        mu_hbm.at[pl.ds(gh, gh), :], mu_v.at[pl.ds(gh, gh), :], sem_mu.at[1])
    cp_x.start()
    cp_mu0.start()
    cp_mu1.start()

    cp_x.wait()
    x = x_v[...]
    y = -2.0 * x
    xsq = x * x
    ones = jnp.ones((G, _LANES), jnp.float32)
    xn = jnp.dot(xsq, ones, preferred_element_type=jnp.float32)

    cp_mu0.wait()
    mu0 = mu_v[0:gh, :]
    dots_v[...] = jnp.dot(y[:, 0:gh], mu0,
                          preferred_element_type=jnp.float32)
    nsq = jnp.sum(mu0 * mu0, axis=0, keepdims=True)

    cp_mu1.wait()
    mu1 = mu_v[gh:G, :]
    nsq = nsq + jnp.sum(mu1 * mu1, axis=0, keepdims=True)

    m = None
    for j in range(K // _KBLOCK):
        blk = slice(j * _KBLOCK, (j + 1) * _KBLOCK)
        d = jnp.dot(y[:, gh:G], mu1[:, blk],
                    preferred_element_type=jnp.float32)
        d = d + dots_v[:, blk]
        for t in range(_KBLOCK // _LANES):
            lo = j * _KBLOCK + t * _LANES
            s = d[:, t * _LANES:(t + 1) * _LANES] + nsq[:, lo:lo + _LANES]
            m = s if m is None else jnp.minimum(m, s)

    tot = m + xn
    out_ref[...] = jnp.min(tot.T, axis=0) * (1.0 / G)


def kernel(images, mu):
    B, G = images.shape
    K = mu.shape[1]
    return pl.pallas_call(
        _loss_kernel,
        in_specs=[
            pl.BlockSpec(memory_space=pltpu.MemorySpace.HBM),
            pl.BlockSpec(memory_space=pltpu.MemorySpace.HBM),
        ],
        out_specs=pl.BlockSpec(memory_space=pltpu.MemorySpace.VMEM),
        out_shape=jax.ShapeDtypeStruct((B,), jnp.float32),
        scratch_shapes=[
            pltpu.VMEM((B, G), jnp.float32),
            pltpu.VMEM((G, K), jnp.float32),
            pltpu.VMEM((B, K), jnp.float32),
            pltpu.SemaphoreType.DMA,
            pltpu.SemaphoreType.DMA((2,)),
        ],
    )(images, mu)


# speedup vs baseline: 1.2503x; 1.2503x over previous
import jax
import jax.numpy as jnp
from jax.experimental import pallas as pl
from jax.experimental.pallas import tpu as pltpu

_KBLOCK = 512
_LANES = 128


def _loss_kernel(x_hbm, mu_hbm, out_ref, x_v, mu_v, sem_x, sem_mu):
    B, G = x_v.shape
    K = mu_v.shape[1]

    cp_x = pltpu.make_async_copy(x_hbm, x_v, sem_x)
    cp_mu = pltpu.make_async_copy(mu_hbm, mu_v, sem_mu)
    cp_x.start()
    cp_mu.start()

    cp_x.wait()
    x = x_v[...]
    y = -2.0 * x
    xsq = x * x
    ones = jnp.ones((G, _LANES), jnp.float32)
    xn = jnp.dot(xsq, ones, preferred_element_type=jnp.float32)

    cp_mu.wait()
    mu = mu_v[...]
    nsq = jnp.sum(mu * mu, axis=0, keepdims=True)

    m = None
    for j in range(K // _KBLOCK):
        blk = slice(j * _KBLOCK, (j + 1) * _KBLOCK)
        d = jnp.dot(y[:, :], mu[:, blk],
                    preferred_element_type=jnp.float32)
        for t in range(_KBLOCK // _LANES):
            lo = j * _KBLOCK + t * _LANES
            s = d[:, t * _LANES:(t + 1) * _LANES] + nsq[:, lo:lo + _LANES]
            m = s if m is None else jnp.minimum(m, s)

    tot = m + xn
    out_ref[...] = jnp.min(tot.T, axis=0) * (1.0 / G)


def kernel(images, mu):
    B, G = images.shape
    K = mu.shape[1]
    return pl.pallas_call(
        _loss_kernel,
        in_specs=[
            pl.BlockSpec(memory_space=pltpu.MemorySpace.HBM),
            pl.BlockSpec(memory_space=pltpu.MemorySpace.HBM),
        ],
        out_specs=pl.BlockSpec(memory_space=pltpu.MemorySpace.VMEM),
        out_shape=jax.ShapeDtypeStruct((B,), jnp.float32),
        scratch_shapes=[
            pltpu.VMEM((B, G), jnp.float32),
            pltpu.VMEM((G, K), jnp.float32),
            pltpu.SemaphoreType.DMA,
            pltpu.SemaphoreType.DMA,
        ],
    )(images, mu)
